# baseline (speedup 1.0000x reference)
import functools

import numpy as np
import jax
import jax.numpy as jnp
from jax import lax
from jax.experimental import pallas as pl
from jax.experimental.pallas import tpu as pltpu
from jax.experimental.pallas import tpu_sc as plsc

_B, _G = 128, 2048
_K = int(0.6 * _G)
_NW = 32
_ROWS_PER_W = _B // _NW
_CHUNKS_PER_ROW = _G // 16

_KS0 = np.uint32(0)
_KS1 = np.uint32(42)
_KS2 = np.uint32(int(_KS0) ^ int(_KS1) ^ 0x1BD11BDA)
_KS = (_KS0, _KS1, _KS2)
_ROT = ((13, 15, 26, 6), (17, 29, 16, 24))

_UNROLL = 4
_GEN_BLOCK_ROWS = 8


def _threefry_keys(pos_u32):
  x0 = jnp.zeros_like(pos_u32)
  x1 = pos_u32 + _KS1
  for i in range(5):
    for r in _ROT[i % 2]:
      x0 = x0 + x1
      x1 = (x1 << np.uint32(r)) | (x1 >> np.uint32(32 - r))
      x1 = x1 ^ x0
    x0 = x0 + _KS[(i + 1) % 3]
    x1 = x1 + np.uint32(int(_KS[(i + 2) % 3]) + i + 1)
  return lax.bitcast_convert_type((x0 ^ x1) >> np.uint32(9), jnp.int32)


def _gen_body(out_ref):
  i = pl.program_id(0)
  base = i * _GEN_BLOCK_ROWS * _G
  pos = (base
         + lax.broadcasted_iota(jnp.int32, (_GEN_BLOCK_ROWS, _G), 0) * _G
         + lax.broadcasted_iota(jnp.int32, (_GEN_BLOCK_ROWS, _G), 1))
  out_ref[...] = _threefry_keys(lax.bitcast_convert_type(pos, jnp.uint32))


@jax.jit
def _gen_keys():
  return pl.pallas_call(
      _gen_body,
      grid=(_B // _GEN_BLOCK_ROWS,),
      out_specs=pl.BlockSpec((_GEN_BLOCK_ROWS, _G), lambda i: (i, 0)),
      out_shape=jax.ShapeDtypeStruct((_B, _G), jnp.int32),
  )()


@functools.lru_cache(maxsize=1)
def _build_select_kernel():
  mesh = plsc.VectorSubcoreMesh(core_axis_name="c", subcore_axis_name="s")
  num_cores = mesh.num_cores

  @functools.partial(
      pl.kernel,
      out_type=jax.ShapeDtypeStruct((_NW, _ROWS_PER_W, 16), jnp.int32),
      mesh=mesh,
      compiler_params=pltpu.CompilerParams(needs_layout_passes=False),
      scratch_types=[
          pltpu.VMEM((_ROWS_PER_W, _G), jnp.int32),
          pltpu.VMEM((4096,), jnp.int32),
          pltpu.VMEM((256,), jnp.int32),
          pltpu.VMEM((2048,), jnp.int32),
          pltpu.VMEM((_ROWS_PER_W, 16), jnp.int32),
          pltpu.SemaphoreType.DMA,
      ],
  )
  def select_kernel(keys_hbm, out_hbm, vbuf, hist, accbuf, candbuf, tbuf, dsem):
    wid = lax.axis_index("s") * num_cores + lax.axis_index("c")
    lane = lax.iota(jnp.int32, 16)

    copy = pltpu.async_copy(
        keys_hbm.at[pl.ds(wid * _ROWS_PER_W, _ROWS_PER_W)], vbuf, dsem)

    ones16 = jnp.full((16,), 1, jnp.int32)
    zeros16 = jnp.zeros((16,), jnp.int32)
    big = jnp.int32(1 << 30)
    big16 = jnp.full((16,), 1 << 30, jnp.int32)

    for cc in range(256):
      hist[pl.ds(cc * 16, 16)] = zeros16
    copy.wait()

    def scan_buckets(nb, k_needed):
      totv = zeros16
      for cc in range(nb // 16):
        parts = [hist[pl.ds(l * nb + cc * 16, 16)] for l in range(16)]
        for l in range(16):
          hist[pl.ds(l * nb + cc * 16, 16)] = zeros16
        while len(parts) > 1:
          parts = [parts[a] + parts[a + 1] for a in range(0, len(parts), 2)]
        acc = parts[0]
        accbuf[pl.ds(cc * 16, 16)] = acc
        totv = jnp.where(lane == cc, jnp.sum(acc), totv)
      cumv = plsc.cumsum(totv)
      chunk_crossed = cumv >= k_needed
      ccross = jnp.min(jnp.where(chunk_crossed, lane, big16))
      carry = jnp.min(jnp.where(lane == ccross, cumv - totv, big16))
      acc = accbuf[pl.ds(ccross * 16, 16)]
      inc = plsc.cumsum(acc) + carry
      crossed = inc >= k_needed
      pstar = jnp.min(jnp.where(crossed, ccross * 16 + lane, big16))
      cb = jnp.min(jnp.where(crossed, inc - acc, big16))
      cnt = jnp.min(jnp.where(crossed, inc, big16)) - cb
      return pstar, cb, cnt

    def select_row(r, carry):
      def scat_pass(shift, nbits, prefix):
        nb = 1 << nbits
        hi_shift = shift + nbits
        unroll = 8

        def scat(ii, i_carry):
          for j in range(unroll):
            v = vbuf[r, pl.ds((ii * unroll + j) * 16, 16)]
            active = (v >> hi_shift) == prefix
            bucket = (v >> shift) & (nb - 1)
            plsc.addupdate_scatter(
                hist, [lane * nb + bucket], ones16, mask=active)
          return i_carry

        lax.fori_loop(0, _CHUNKS_PER_ROW // unroll, scat, jnp.int32(0))

      scat_pass(15, 8, jnp.int32(0))
      p1, cb1, cnt1 = scan_buckets(256, jnp.int32(_K))
      k2 = jnp.int32(_K) - cb1

      @pl.when(cnt1 <= 16)
      def _sort_path():
        candbuf[pl.ds(0, 16)] = big16

        cunroll = 8

        def compact(ii, off):
          for j in range(cunroll):
            i = ii * cunroll + j
            v = vbuf[r, pl.ds(i * 16, 16)]
            m = (v >> 15) == p1
            mi = jnp.where(m, 1, 0)
            pc = plsc.cumsum(mi)
            comp = ((v & 0x7FFF) << 11) | (i * 16 + lane)
            plsc.store_scatter(candbuf, [off + pc - 1], comp, mask=m)
            off = off + plsc.all_reduce_population_count(m)
          return off

        lax.fori_loop(0, _CHUNKS_PER_ROW // cunroll, compact, zeros16)
        cand = candbuf[pl.ds(0, 16)]
        skey, _ = plsc.sort_key_val(cand, cand)
        csel = jnp.min(jnp.where(lane == k2 - 1, skey, big16))
        t = (p1 << 15) | (csel >> 11)
        jcut = csel & 0x7FF
        tbuf[r, pl.ds(0, 16)] = jnp.where(
            lane == 0, t, jnp.where(lane == 1, jcut, 0))

      @pl.when(cnt1 > 16)
      def _hist_path():
        prefix = p1
        k_needed = k2
        cnt = cnt1
        for shift, nbits in ((7, 8), (0, 7)):
          nb = 1 << nbits
          scat_pass(shift, nbits, prefix)
          pstar, cb, cnt = scan_buckets(nb, k_needed)
          prefix = (prefix << nbits) | pstar
          k_needed = k_needed - cb
        t = prefix

        @pl.when(k_needed == cnt)
        def _take_all():
          tbuf[r, pl.ds(0, 16)] = jnp.where(
              lane == 0, t, jnp.where(lane == 1, jnp.int32(_G), 0))

        @pl.when(k_needed != cnt)
        def _split():
          def fin(i, state):
            eqc, jcand = state
            v = vbuf[r, pl.ds(i * 16, 16)]
            eqi = jnp.where(v == t, 1, 0)
            rank = plsc.cumsum(eqi) + eqc
            hit = (eqi > 0) & (rank == k_needed)
            jcand = jnp.minimum(
                jcand, jnp.min(jnp.where(hit, i * 16 + lane, big16)))
            return eqc + jnp.sum(eqi), jcand

          _, jcut = lax.fori_loop(0, _CHUNKS_PER_ROW, fin,
                                  (jnp.int32(0), big))
          tbuf[r, pl.ds(0, 16)] = jnp.where(
              lane == 0, t, jnp.where(lane == 1, jcut, 0))

      return carry

    lax.fori_loop(0, _ROWS_PER_W, select_row, jnp.int32(0))

    pltpu.sync_copy(tbuf, out_hbm.at[wid])

  return select_kernel


def _mask_body(keys_ref, trip_ref, out_ref):
  k = keys_ref[...]
  t = trip_ref[:, 0:1]
  jcut = trip_ref[:, 1:2]
  idx = lax.broadcasted_iota(jnp.int32, k.shape, 1)
  m = (k < t) | ((k == t) & (idx <= jcut))
  out_ref[...] = m.astype(jnp.int8)


@jax.jit
def _finalize(keys, trip):
  return pl.pallas_call(
      _mask_body,
      in_specs=[
          pl.BlockSpec((_B, _G), lambda: (0, 0)),
          pl.BlockSpec((_B, 16), lambda: (0, 0)),
      ],
      out_specs=pl.BlockSpec((_B, _G), lambda: (0, 0)),
      out_shape=jax.ShapeDtypeStruct((_B, _G), jnp.int8),
  )(keys, trip).astype(jnp.bool_)


def kernel(x):
  del x
  keys = _gen_keys()
  trip = _build_select_kernel()(keys).reshape(_B, 16)
  return _finalize(keys, trip)


# speedup vs baseline: 1.0699x; 1.0699x over previous
import functools

import numpy as np
import jax
import jax.numpy as jnp
from jax import lax
from jax.experimental import pallas as pl
from jax.experimental.pallas import tpu as pltpu
from jax.experimental.pallas import tpu_sc as plsc

_B, _G = 128, 2048
_K = int(0.6 * _G)
_NW = 32
_ROWS_PER_W = _B // _NW
_CHUNKS_PER_ROW = _G // 16

_KS0 = np.uint32(0)
_KS1 = np.uint32(42)
_KS2 = np.uint32(int(_KS0) ^ int(_KS1) ^ 0x1BD11BDA)
_KS = (_KS0, _KS1, _KS2)
_ROT = ((13, 15, 26, 6), (17, 29, 16, 24))

_UNROLL = 4
_GEN_BLOCK_ROWS = 32


def _threefry_keys(pos_u32):
  x0 = jnp.zeros_like(pos_u32)
  x1 = pos_u32 + _KS1
  for i in range(5):
    for r in _ROT[i % 2]:
      x0 = x0 + x1
      x1 = (x1 << np.uint32(r)) | (x1 >> np.uint32(32 - r))
      x1 = x1 ^ x0
    x0 = x0 + _KS[(i + 1) % 3]
    x1 = x1 + np.uint32(int(_KS[(i + 2) % 3]) + i + 1)
  return lax.bitcast_convert_type((x0 ^ x1) >> np.uint32(9), jnp.int32)


def _gen_body(out_ref):
  i = pl.program_id(0)
  base = i * _GEN_BLOCK_ROWS * _G
  pos = (base
         + lax.broadcasted_iota(jnp.int32, (_GEN_BLOCK_ROWS, _G), 0) * _G
         + lax.broadcasted_iota(jnp.int32, (_GEN_BLOCK_ROWS, _G), 1))
  out_ref[...] = _threefry_keys(lax.bitcast_convert_type(pos, jnp.uint32))


@jax.jit
def _gen_keys():
  return pl.pallas_call(
      _gen_body,
      grid=(_B // _GEN_BLOCK_ROWS,),
      out_specs=pl.BlockSpec((_GEN_BLOCK_ROWS, _G), lambda i: (i, 0)),
      out_shape=jax.ShapeDtypeStruct((_B, _G), jnp.int32),
  )()


@functools.lru_cache(maxsize=1)
def _build_select_kernel():
  mesh = plsc.VectorSubcoreMesh(core_axis_name="c", subcore_axis_name="s")
  num_cores = mesh.num_cores

  @functools.partial(
      pl.kernel,
      out_type=jax.ShapeDtypeStruct((_NW, _ROWS_PER_W, 16), jnp.int32),
      mesh=mesh,
      compiler_params=pltpu.CompilerParams(needs_layout_passes=False),
      scratch_types=[
          pltpu.VMEM((_ROWS_PER_W, _G), jnp.int32),
          pltpu.VMEM((4096,), jnp.int32),
          pltpu.VMEM((256,), jnp.int32),
          pltpu.VMEM((2048,), jnp.int32),
          pltpu.VMEM((_ROWS_PER_W, 16), jnp.int32),
          pltpu.SemaphoreType.DMA,
      ],
  )
  def select_kernel(keys_hbm, out_hbm, vbuf, hist, accbuf, candbuf, tbuf, dsem):
    wid = lax.axis_index("s") * num_cores + lax.axis_index("c")
    lane = lax.iota(jnp.int32, 16)

    copy = pltpu.async_copy(
        keys_hbm.at[pl.ds(wid * _ROWS_PER_W, _ROWS_PER_W)], vbuf, dsem)

    ones16 = jnp.full((16,), 1, jnp.int32)
    zeros16 = jnp.zeros((16,), jnp.int32)
    big = jnp.int32(1 << 30)
    big16 = jnp.full((16,), 1 << 30, jnp.int32)

    for cc in range(256):
      hist[pl.ds(cc * 16, 16)] = zeros16
    copy.wait()

    def scan_buckets(nb, k_needed):
      totv = zeros16
      for cc in range(nb // 16):
        parts = [hist[pl.ds(l * nb + cc * 16, 16)] for l in range(16)]
        for l in range(16):
          hist[pl.ds(l * nb + cc * 16, 16)] = zeros16
        while len(parts) > 1:
          parts = [parts[a] + parts[a + 1] for a in range(0, len(parts), 2)]
        acc = parts[0]
        accbuf[pl.ds(cc * 16, 16)] = acc
        totv = jnp.where(lane == cc, jnp.sum(acc), totv)
      cumv = plsc.cumsum(totv)
      chunk_crossed = cumv >= k_needed
      ccross = jnp.min(jnp.where(chunk_crossed, lane, big16))
      carry = jnp.min(jnp.where(lane == ccross, cumv - totv, big16))
      acc = accbuf[pl.ds(ccross * 16, 16)]
      inc = plsc.cumsum(acc) + carry
      crossed = inc >= k_needed
      pstar = jnp.min(jnp.where(crossed, ccross * 16 + lane, big16))
      cb = jnp.min(jnp.where(crossed, inc - acc, big16))
      cnt = jnp.min(jnp.where(crossed, inc, big16)) - cb
      return pstar, cb, cnt

    def select_row(r, carry):
      def scat_pass(shift, nbits, prefix):
        nb = 1 << nbits
        hi_shift = shift + nbits
        unroll = 8

        def scat(ii, i_carry):
          for j in range(unroll):
            v = vbuf[r, pl.ds((ii * unroll + j) * 16, 16)]
            active = (v >> hi_shift) == prefix
            bucket = (v >> shift) & (nb - 1)
            plsc.addupdate_scatter(
                hist, [lane * nb + bucket], ones16, mask=active)
          return i_carry

        lax.fori_loop(0, _CHUNKS_PER_ROW // unroll, scat, jnp.int32(0))

      scat_pass(15, 8, jnp.int32(0))
      p1, cb1, cnt1 = scan_buckets(256, jnp.int32(_K))
      k2 = jnp.int32(_K) - cb1

      @pl.when(cnt1 <= 16)
      def _sort_path():
        candbuf[pl.ds(0, 16)] = big16

        cunroll = 8

        def compact(ii, off):
          for j in range(cunroll):
            i = ii * cunroll + j
            v = vbuf[r, pl.ds(i * 16, 16)]
            m = (v >> 15) == p1
            mi = jnp.where(m, 1, 0)
            pc = plsc.cumsum(mi)
            comp = ((v & 0x7FFF) << 11) | (i * 16 + lane)
            plsc.store_scatter(candbuf, [off + pc - 1], comp, mask=m)
            off = off + plsc.all_reduce_population_count(m)
          return off

        lax.fori_loop(0, _CHUNKS_PER_ROW // cunroll, compact, zeros16)
        cand = candbuf[pl.ds(0, 16)]
        skey, _ = plsc.sort_key_val(cand, cand)
        csel = jnp.min(jnp.where(lane == k2 - 1, skey, big16))
        t = (p1 << 15) | (csel >> 11)
        jcut = csel & 0x7FF
        tbuf[r, pl.ds(0, 16)] = jnp.where(
            lane == 0, t, jnp.where(lane == 1, jcut, 0))

      @pl.when(cnt1 > 16)
      def _hist_path():
        prefix = p1
        k_needed = k2
        cnt = cnt1
        for shift, nbits in ((7, 8), (0, 7)):
          nb = 1 << nbits
          scat_pass(shift, nbits, prefix)
          pstar, cb, cnt = scan_buckets(nb, k_needed)
          prefix = (prefix << nbits) | pstar
          k_needed = k_needed - cb
        t = prefix

        @pl.when(k_needed == cnt)
        def _take_all():
          tbuf[r, pl.ds(0, 16)] = jnp.where(
              lane == 0, t, jnp.where(lane == 1, jnp.int32(_G), 0))

        @pl.when(k_needed != cnt)
        def _split():
          def fin(i, state):
            eqc, jcand = state
            v = vbuf[r, pl.ds(i * 16, 16)]
            eqi = jnp.where(v == t, 1, 0)
            rank = plsc.cumsum(eqi) + eqc
            hit = (eqi > 0) & (rank == k_needed)
            jcand = jnp.minimum(
                jcand, jnp.min(jnp.where(hit, i * 16 + lane, big16)))
            return eqc + jnp.sum(eqi), jcand

          _, jcut = lax.fori_loop(0, _CHUNKS_PER_ROW, fin,
                                  (jnp.int32(0), big))
          tbuf[r, pl.ds(0, 16)] = jnp.where(
              lane == 0, t, jnp.where(lane == 1, jcut, 0))

      return carry

    lax.fori_loop(0, _ROWS_PER_W, select_row, jnp.int32(0))

    pltpu.sync_copy(tbuf, out_hbm.at[wid])

  return select_kernel


def _mask_body(keys_ref, trip_ref, out_ref):
  k = keys_ref[...]
  t = trip_ref[:, 0:1]
  jcut = trip_ref[:, 1:2]
  idx = lax.broadcasted_iota(jnp.int32, k.shape, 1)
  m = (k < t) | ((k == t) & (idx <= jcut))
  out_ref[...] = m.astype(jnp.int8)


@jax.jit
def _finalize(keys, trip):
  return pl.pallas_call(
      _mask_body,
      in_specs=[
          pl.BlockSpec((_B, _G), lambda: (0, 0)),
          pl.BlockSpec((_B, 16), lambda: (0, 0)),
      ],
      out_specs=pl.BlockSpec((_B, _G), lambda: (0, 0)),
      out_shape=jax.ShapeDtypeStruct((_B, _G), jnp.int8),
  )(keys, trip).astype(jnp.bool_)


def kernel(x):
  del x
  keys = _gen_keys()
  trip = _build_select_kernel()(keys).reshape(_B, 16)
  return _finalize(keys, trip)


# speedup vs baseline: 1.0714x; 1.0014x over previous
import functools

import numpy as np
import jax
import jax.numpy as jnp
from jax import lax
from jax.experimental import pallas as pl
from jax.experimental.pallas import tpu as pltpu
from jax.experimental.pallas import tpu_sc as plsc

_B, _G = 128, 2048
_K = int(0.6 * _G)
_NW = 32
_ROWS_PER_W = _B // _NW
_CHUNKS_PER_ROW = _G // 16

_KS0 = np.uint32(0)
_KS1 = np.uint32(42)
_KS2 = np.uint32(int(_KS0) ^ int(_KS1) ^ 0x1BD11BDA)
_KS = (_KS0, _KS1, _KS2)
_ROT = ((13, 15, 26, 6), (17, 29, 16, 24))

_UNROLL = 4
_GEN_BLOCK_ROWS = 64


def _threefry_keys(pos_u32):
  x0 = jnp.zeros_like(pos_u32)
  x1 = pos_u32 + _KS1
  for i in range(5):
    for r in _ROT[i % 2]:
      x0 = x0 + x1
      x1 = (x1 << np.uint32(r)) | (x1 >> np.uint32(32 - r))
      x1 = x1 ^ x0
    x0 = x0 + _KS[(i + 1) % 3]
    x1 = x1 + np.uint32(int(_KS[(i + 2) % 3]) + i + 1)
  return lax.bitcast_convert_type((x0 ^ x1) >> np.uint32(9), jnp.int32)


def _gen_body(out_ref):
  i = pl.program_id(0)
  base = i * _GEN_BLOCK_ROWS * _G
  pos = (base
         + lax.broadcasted_iota(jnp.int32, (_GEN_BLOCK_ROWS, _G), 0) * _G
         + lax.broadcasted_iota(jnp.int32, (_GEN_BLOCK_ROWS, _G), 1))
  out_ref[...] = _threefry_keys(lax.bitcast_convert_type(pos, jnp.uint32))


@jax.jit
def _gen_keys():
  return pl.pallas_call(
      _gen_body,
      grid=(_B // _GEN_BLOCK_ROWS,),
      out_specs=pl.BlockSpec((_GEN_BLOCK_ROWS, _G), lambda i: (i, 0)),
      out_shape=jax.ShapeDtypeStruct((_B, _G), jnp.int32),
  )()


@functools.lru_cache(maxsize=1)
def _build_select_kernel():
  mesh = plsc.VectorSubcoreMesh(core_axis_name="c", subcore_axis_name="s")
  num_cores = mesh.num_cores

  @functools.partial(
      pl.kernel,
      out_type=jax.ShapeDtypeStruct((_NW, _ROWS_PER_W, 16), jnp.int32),
      mesh=mesh,
      compiler_params=pltpu.CompilerParams(needs_layout_passes=False),
      scratch_types=[
          pltpu.VMEM((_ROWS_PER_W, _G), jnp.int32),
          pltpu.VMEM((4096,), jnp.int32),
          pltpu.VMEM((256,), jnp.int32),
          pltpu.VMEM((2048,), jnp.int32),
          pltpu.VMEM((_ROWS_PER_W, 16), jnp.int32),
          pltpu.SemaphoreType.DMA,
      ],
  )
  def select_kernel(keys_hbm, out_hbm, vbuf, hist, accbuf, candbuf, tbuf, dsem):
    wid = lax.axis_index("s") * num_cores + lax.axis_index("c")
    lane = lax.iota(jnp.int32, 16)

    copy = pltpu.async_copy(
        keys_hbm.at[pl.ds(wid * _ROWS_PER_W, _ROWS_PER_W)], vbuf, dsem)

    ones16 = jnp.full((16,), 1, jnp.int32)
    zeros16 = jnp.zeros((16,), jnp.int32)
    big = jnp.int32(1 << 30)
    big16 = jnp.full((16,), 1 << 30, jnp.int32)

    for cc in range(256):
      hist[pl.ds(cc * 16, 16)] = zeros16
    copy.wait()

    def scan_buckets(nb, k_needed):
      totv = zeros16
      for cc in range(nb // 16):
        parts = [hist[pl.ds(l * nb + cc * 16, 16)] for l in range(16)]
        for l in range(16):
          hist[pl.ds(l * nb + cc * 16, 16)] = zeros16
        while len(parts) > 1:
          parts = [parts[a] + parts[a + 1] for a in range(0, len(parts), 2)]
        acc = parts[0]
        accbuf[pl.ds(cc * 16, 16)] = acc
        totv = jnp.where(lane == cc, jnp.sum(acc), totv)
      cumv = plsc.cumsum(totv)
      chunk_crossed = cumv >= k_needed
      ccross = jnp.min(jnp.where(chunk_crossed, lane, big16))
      carry = jnp.min(jnp.where(lane == ccross, cumv - totv, big16))
      acc = accbuf[pl.ds(ccross * 16, 16)]
      inc = plsc.cumsum(acc) + carry
      crossed = inc >= k_needed
      pstar = jnp.min(jnp.where(crossed, ccross * 16 + lane, big16))
      cb = jnp.min(jnp.where(crossed, inc - acc, big16))
      cnt = jnp.min(jnp.where(crossed, inc, big16)) - cb
      return pstar, cb, cnt

    def select_row(r, carry):
      def scat_pass(shift, nbits, prefix):
        nb = 1 << nbits
        hi_shift = shift + nbits
        unroll = 8

        def scat(ii, i_carry):
          for j in range(unroll):
            v = vbuf[r, pl.ds((ii * unroll + j) * 16, 16)]
            active = (v >> hi_shift) == prefix
            bucket = (v >> shift) & (nb - 1)
            plsc.addupdate_scatter(
                hist, [lane * nb + bucket], ones16, mask=active)
          return i_carry

        lax.fori_loop(0, _CHUNKS_PER_ROW // unroll, scat, jnp.int32(0))

      scat_pass(15, 8, jnp.int32(0))
      p1, cb1, cnt1 = scan_buckets(256, jnp.int32(_K))
      k2 = jnp.int32(_K) - cb1

      @pl.when(cnt1 <= 16)
      def _sort_path():
        candbuf[pl.ds(0, 16)] = big16

        cunroll = 8

        def compact(ii, off):
          for j in range(cunroll):
            i = ii * cunroll + j
            v = vbuf[r, pl.ds(i * 16, 16)]
            m = (v >> 15) == p1
            mi = jnp.where(m, 1, 0)
            pc = plsc.cumsum(mi)
            comp = ((v & 0x7FFF) << 11) | (i * 16 + lane)
            plsc.store_scatter(candbuf, [off + pc - 1], comp, mask=m)
            off = off + plsc.all_reduce_population_count(m)
          return off

        lax.fori_loop(0, _CHUNKS_PER_ROW // cunroll, compact, zeros16)
        cand = candbuf[pl.ds(0, 16)]
        skey, _ = plsc.sort_key_val(cand, cand)
        csel = jnp.min(jnp.where(lane == k2 - 1, skey, big16))
        t = (p1 << 15) | (csel >> 11)
        jcut = csel & 0x7FF
        tbuf[r, pl.ds(0, 16)] = jnp.where(
            lane == 0, t, jnp.where(lane == 1, jcut, 0))

      @pl.when(cnt1 > 16)
      def _hist_path():
        prefix = p1
        k_needed = k2
        cnt = cnt1
        for shift, nbits in ((7, 8), (0, 7)):
          nb = 1 << nbits
          scat_pass(shift, nbits, prefix)
          pstar, cb, cnt = scan_buckets(nb, k_needed)
          prefix = (prefix << nbits) | pstar
          k_needed = k_needed - cb
        t = prefix

        @pl.when(k_needed == cnt)
        def _take_all():
          tbuf[r, pl.ds(0, 16)] = jnp.where(
              lane == 0, t, jnp.where(lane == 1, jnp.int32(_G), 0))

        @pl.when(k_needed != cnt)
        def _split():
          def fin(i, state):
            eqc, jcand = state
            v = vbuf[r, pl.ds(i * 16, 16)]
            eqi = jnp.where(v == t, 1, 0)
            rank = plsc.cumsum(eqi) + eqc
            hit = (eqi > 0) & (rank == k_needed)
            jcand = jnp.minimum(
                jcand, jnp.min(jnp.where(hit, i * 16 + lane, big16)))
            return eqc + jnp.sum(eqi), jcand

          _, jcut = lax.fori_loop(0, _CHUNKS_PER_ROW, fin,
                                  (jnp.int32(0), big))
          tbuf[r, pl.ds(0, 16)] = jnp.where(
              lane == 0, t, jnp.where(lane == 1, jcut, 0))

      return carry

    lax.fori_loop(0, _ROWS_PER_W, select_row, jnp.int32(0))

    pltpu.sync_copy(tbuf, out_hbm.at[wid])

  return select_kernel


def _mask_body(keys_ref, trip_ref, out_ref):
  k = keys_ref[...]
  t = trip_ref[:, 0:1]
  jcut = trip_ref[:, 1:2]
  idx = lax.broadcasted_iota(jnp.int32, k.shape, 1)
  m = (k < t) | ((k == t) & (idx <= jcut))
  out_ref[...] = m.astype(jnp.int8)


@jax.jit
def _finalize(keys, trip):
  return pl.pallas_call(
      _mask_body,
      in_specs=[
          pl.BlockSpec((_B, _G), lambda: (0, 0)),
          pl.BlockSpec((_B, 16), lambda: (0, 0)),
      ],
      out_specs=pl.BlockSpec((_B, _G), lambda: (0, 0)),
      out_shape=jax.ShapeDtypeStruct((_B, _G), jnp.int8),
  )(keys, trip).astype(jnp.bool_)


def kernel(x):
  del x
  keys = _gen_keys()
  trip = _build_select_kernel()(keys).reshape(_B, 16)
  return _finalize(keys, trip)


# speedup vs baseline: 1.0715x; 1.0000x over previous
import functools

import numpy as np
import jax
import jax.numpy as jnp
from jax import lax
from jax.experimental import pallas as pl
from jax.experimental.pallas import tpu as pltpu
from jax.experimental.pallas import tpu_sc as plsc

_B, _G = 128, 2048
_K = int(0.6 * _G)
_NW = 32
_ROWS_PER_W = _B // _NW
_CHUNKS_PER_ROW = _G // 16

_KS0 = np.uint32(0)
_KS1 = np.uint32(42)
_KS2 = np.uint32(int(_KS0) ^ int(_KS1) ^ 0x1BD11BDA)
_KS = (_KS0, _KS1, _KS2)
_ROT = ((13, 15, 26, 6), (17, 29, 16, 24))

_UNROLL = 4
_GEN_BLOCK_ROWS = 64


def _threefry_keys(pos_u32):
  x0 = jnp.zeros_like(pos_u32)
  x1 = pos_u32 + _KS1
  for i in range(5):
    for r in _ROT[i % 2]:
      x0 = x0 + x1
      x1 = (x1 << np.uint32(r)) | (x1 >> np.uint32(32 - r))
      x1 = x1 ^ x0
    x0 = x0 + _KS[(i + 1) % 3]
    x1 = x1 + np.uint32(int(_KS[(i + 2) % 3]) + i + 1)
  return lax.bitcast_convert_type((x0 ^ x1) >> np.uint32(9), jnp.int32)


def _gen_body(out_ref):
  i = pl.program_id(0)
  base = i * _GEN_BLOCK_ROWS * _G
  pos = (base
         + lax.broadcasted_iota(jnp.int32, (_GEN_BLOCK_ROWS, _G), 0) * _G
         + lax.broadcasted_iota(jnp.int32, (_GEN_BLOCK_ROWS, _G), 1))
  out_ref[...] = _threefry_keys(lax.bitcast_convert_type(pos, jnp.uint32))


@jax.jit
def _gen_keys():
  return pl.pallas_call(
      _gen_body,
      grid=(_B // _GEN_BLOCK_ROWS,),
      out_specs=pl.BlockSpec((_GEN_BLOCK_ROWS, _G), lambda i: (i, 0)),
      out_shape=jax.ShapeDtypeStruct((_B, _G), jnp.int32),
  )()


@functools.lru_cache(maxsize=1)
def _build_select_kernel():
  mesh = plsc.VectorSubcoreMesh(core_axis_name="c", subcore_axis_name="s")
  num_cores = mesh.num_cores

  @functools.partial(
      pl.kernel,
      out_type=jax.ShapeDtypeStruct((_NW, _ROWS_PER_W, 16), jnp.int32),
      mesh=mesh,
      compiler_params=pltpu.CompilerParams(needs_layout_passes=False),
      scratch_types=[
          pltpu.VMEM((_ROWS_PER_W, _G), jnp.int32),
          pltpu.VMEM((4096,), jnp.int32),
          pltpu.VMEM((256,), jnp.int32),
          pltpu.VMEM((2048,), jnp.int32),
          pltpu.VMEM((_ROWS_PER_W, 16), jnp.int32),
          pltpu.SemaphoreType.DMA,
      ],
  )
  def select_kernel(keys_hbm, out_hbm, vbuf, hist, accbuf, candbuf, tbuf, dsem):
    wid = lax.axis_index("s") * num_cores + lax.axis_index("c")
    lane = lax.iota(jnp.int32, 16)

    copy = pltpu.async_copy(
        keys_hbm.at[pl.ds(wid * _ROWS_PER_W, _ROWS_PER_W)], vbuf, dsem)

    ones16 = jnp.full((16,), 1, jnp.int32)
    zeros16 = jnp.zeros((16,), jnp.int32)
    big = jnp.int32(1 << 30)
    big16 = jnp.full((16,), 1 << 30, jnp.int32)

    for cc in range(256):
      hist[pl.ds(cc * 16, 16)] = zeros16
    copy.wait()

    def scan_buckets(nb, k_needed):
      totv = zeros16
      for cc in range(nb // 16):
        parts = [hist[pl.ds(l * nb + cc * 16, 16)] for l in range(16)]
        for l in range(16):
          hist[pl.ds(l * nb + cc * 16, 16)] = zeros16
        while len(parts) > 1:
          parts = [parts[a] + parts[a + 1] for a in range(0, len(parts), 2)]
        acc = parts[0]
        accbuf[pl.ds(cc * 16, 16)] = acc
        totv = jnp.where(lane == cc, jnp.sum(acc), totv)
      cumv = plsc.cumsum(totv)
      chunk_crossed = cumv >= k_needed
      ccross = jnp.min(jnp.where(chunk_crossed, lane, big16))
      carry = jnp.min(jnp.where(lane == ccross, cumv - totv, big16))
      acc = accbuf[pl.ds(ccross * 16, 16)]
      inc = plsc.cumsum(acc) + carry
      crossed = inc >= k_needed
      pstar = jnp.min(jnp.where(crossed, ccross * 16 + lane, big16))
      cb = jnp.min(jnp.where(crossed, inc - acc, big16))
      cnt = jnp.min(jnp.where(crossed, inc, big16)) - cb
      return pstar, cb, cnt

    def select_row(r, carry):
      def scat_pass(shift, nbits, prefix):
        nb = 1 << nbits
        hi_shift = shift + nbits
        unroll = 8

        def scat(ii, i_carry):
          for j in range(unroll):
            v = vbuf[r, pl.ds((ii * unroll + j) * 16, 16)]
            active = (v >> hi_shift) == prefix
            bucket = (v >> shift) & (nb - 1)
            plsc.addupdate_scatter(
                hist, [lane * nb + bucket], ones16, mask=active)
          return i_carry

        lax.fori_loop(0, _CHUNKS_PER_ROW // unroll, scat, jnp.int32(0))

      scat_pass(15, 8, jnp.int32(0))
      p1, cb1, cnt1 = scan_buckets(256, jnp.int32(_K))
      k2 = jnp.int32(_K) - cb1

      @pl.when(cnt1 <= 16)
      def _sort_path():
        candbuf[pl.ds(0, 16)] = big16

        cunroll = 8

        def compact(ii, off):
          for j in range(cunroll):
            i = ii * cunroll + j
            v = vbuf[r, pl.ds(i * 16, 16)]
            m = (v >> 15) == p1
            mi = jnp.where(m, 1, 0)
            pc = plsc.cumsum(mi)
            comp = ((v & 0x7FFF) << 11) | (i * 16 + lane)
            plsc.store_scatter(candbuf, [off + pc - 1], comp, mask=m)
            off = off + plsc.all_reduce_population_count(m)
          return off

        lax.fori_loop(0, _CHUNKS_PER_ROW // cunroll, compact, zeros16)
        cand = candbuf[pl.ds(0, 16)]
        skey, _ = plsc.sort_key_val(cand, cand)
        csel = jnp.min(jnp.where(lane == k2 - 1, skey, big16))
        t = (p1 << 15) | (csel >> 11)
        jcut = csel & 0x7FF
        tbuf[r, pl.ds(0, 16)] = jnp.where(
            lane == 0, t, jnp.where(lane == 1, jcut, 0))

      @pl.when(cnt1 > 16)
      def _hist_path():
        prefix = p1
        k_needed = k2
        cnt = cnt1
        for shift, nbits in ((7, 8), (0, 7)):
          nb = 1 << nbits
          scat_pass(shift, nbits, prefix)
          pstar, cb, cnt = scan_buckets(nb, k_needed)
          prefix = (prefix << nbits) | pstar
          k_needed = k_needed - cb
        t = prefix

        @pl.when(k_needed == cnt)
        def _take_all():
          tbuf[r, pl.ds(0, 16)] = jnp.where(
              lane == 0, t, jnp.where(lane == 1, jnp.int32(_G), 0))

        @pl.when(k_needed != cnt)
        def _split():
          def fin(i, state):
            eqc, jcand = state
            v = vbuf[r, pl.ds(i * 16, 16)]
            eqi = jnp.where(v == t, 1, 0)
            rank = plsc.cumsum(eqi) + eqc
            hit = (eqi > 0) & (rank == k_needed)
            jcand = jnp.minimum(
                jcand, jnp.min(jnp.where(hit, i * 16 + lane, big16)))
            return eqc + jnp.sum(eqi), jcand

          _, jcut = lax.fori_loop(0, _CHUNKS_PER_ROW, fin,
                                  (jnp.int32(0), big))
          tbuf[r, pl.ds(0, 16)] = jnp.where(
              lane == 0, t, jnp.where(lane == 1, jcut, 0))

      return carry

    lax.fori_loop(0, _ROWS_PER_W, select_row, jnp.int32(0))

    pltpu.sync_copy(tbuf, out_hbm.at[wid])

  return select_kernel


def _mask_body(keys_ref, trip_ref, out_ref):
  k = keys_ref[...]
  t = trip_ref[:, 0:1]
  jcut = trip_ref[:, 1:2]
  idx = lax.broadcasted_iota(jnp.int32, k.shape, 1)
  m = (k < t) | ((k == t) & (idx <= jcut))
  out_ref[...] = m.astype(jnp.int8)


@jax.jit
def _finalize(keys, trip):
  return pl.pallas_call(
      _mask_body,
      grid=(2,),
      in_specs=[
          pl.BlockSpec((_B // 2, _G), lambda i: (i, 0)),
          pl.BlockSpec((_B // 2, 16), lambda i: (i, 0)),
      ],
      out_specs=pl.BlockSpec((_B // 2, _G), lambda i: (i, 0)),
      out_shape=jax.ShapeDtypeStruct((_B, _G), jnp.int8),
  )(keys, trip).astype(jnp.bool_)


def kernel(x):
  del x
  keys = _gen_keys()
  trip = _build_select_kernel()(keys).reshape(_B, 16)
  return _finalize(keys, trip)
